# baseline (speedup 1.0000x reference)
import jax
import jax.numpy as jnp
from jax.experimental import pallas as pl

_N_ITERS = 12
_BLOCK_ROWS = 128


def _sparsemax_rows(x_ref, o_ref):
    x = x_ref[...]
    m = jnp.max(x, axis=1, keepdims=True)
    s_minus_1 = jnp.sum(x, axis=1, keepdims=True) - 1.0
    lo = m - 1.0
    hi = m
    for _ in range(_N_ITERS):
        mid = 0.5 * (lo + hi)
        smin = jnp.sum(jnp.minimum(x, mid), axis=1, keepdims=True)
        pred = smin <= s_minus_1
        lo = jnp.where(pred, mid, lo)
        hi = jnp.where(pred, hi, mid)
    mid = 0.5 * (lo + hi)
    mask = x > mid
    k = jnp.sum(mask.astype(jnp.float32), axis=1, keepdims=True)
    s = jnp.sum(jnp.where(mask, x, 0.0), axis=1, keepdims=True)
    tau = (s - 1.0) / k
    o_ref[...] = jnp.maximum(x - tau, 0.0)


def kernel(input):
    orig_shape = input.shape
    n = orig_shape[-1]
    x = input.reshape(-1, n)
    rows = x.shape[0]
    r = _BLOCK_ROWS if rows % _BLOCK_ROWS == 0 else rows
    out = pl.pallas_call(
        _sparsemax_rows,
        grid=(rows // r,),
        in_specs=[pl.BlockSpec((r, n), lambda i: (i, 0))],
        out_specs=pl.BlockSpec((r, n), lambda i: (i, 0)),
        out_shape=jax.ShapeDtypeStruct((rows, n), x.dtype),
    )(x)
    return out.reshape(orig_shape)


# speedup vs baseline: 1.1519x; 1.1519x over previous
import jax
import jax.numpy as jnp
from jax.experimental import pallas as pl

_N_ITERS = 12
_BLOCK_ROWS = 256


def _sparsemax_rows(x_ref, o_ref):
    x = x_ref[...]
    m = jnp.max(x, axis=1, keepdims=True)
    s_minus_1 = jnp.sum(x, axis=1, keepdims=True) - 1.0
    lo = m - 1.0
    hi = m
    for _ in range(_N_ITERS):
        mid = 0.5 * (lo + hi)
        smin = jnp.sum(jnp.minimum(x, mid), axis=1, keepdims=True)
        pred = smin <= s_minus_1
        lo = jnp.where(pred, mid, lo)
        hi = jnp.where(pred, hi, mid)
    mid = 0.5 * (lo + hi)
    mask = x > mid
    k = jnp.sum(mask.astype(jnp.float32), axis=1, keepdims=True)
    s = jnp.sum(jnp.where(mask, x, 0.0), axis=1, keepdims=True)
    tau = (s - 1.0) / k
    o_ref[...] = jnp.maximum(x - tau, 0.0)


def kernel(input):
    orig_shape = input.shape
    n = orig_shape[-1]
    x = input.reshape(-1, n)
    rows = x.shape[0]
    r = _BLOCK_ROWS if rows % _BLOCK_ROWS == 0 else rows
    out = pl.pallas_call(
        _sparsemax_rows,
        grid=(rows // r,),
        in_specs=[pl.BlockSpec((r, n), lambda i: (i, 0))],
        out_specs=pl.BlockSpec((r, n), lambda i: (i, 0)),
        out_shape=jax.ShapeDtypeStruct((rows, n), x.dtype),
    )(x)
    return out.reshape(orig_shape)


# speedup vs baseline: 1.3006x; 1.1290x over previous
import jax
import jax.numpy as jnp
from jax.experimental import pallas as pl

_N_ITERS = 10
_BLOCK_ROWS = 256


def _sparsemax_rows(x_ref, o_ref):
    x = x_ref[...]
    m = jnp.max(x, axis=1, keepdims=True)
    s_minus_1 = jnp.sum(x, axis=1, keepdims=True) - 1.0
    lo = m - 1.0
    hi = m
    for _ in range(_N_ITERS):
        mid = 0.5 * (lo + hi)
        smin = jnp.sum(jnp.minimum(x, mid), axis=1, keepdims=True)
        pred = smin <= s_minus_1
        lo = jnp.where(pred, mid, lo)
        hi = jnp.where(pred, hi, mid)
    mid = 0.5 * (lo + hi)
    mask = x > mid
    k = jnp.sum(mask.astype(jnp.float32), axis=1, keepdims=True)
    s = jnp.sum(jnp.where(mask, x, 0.0), axis=1, keepdims=True)
    tau = (s - 1.0) / k
    o_ref[...] = jnp.maximum(x - tau, 0.0)


def kernel(input):
    orig_shape = input.shape
    n = orig_shape[-1]
    x = input.reshape(-1, n)
    rows = x.shape[0]
    r = _BLOCK_ROWS if rows % _BLOCK_ROWS == 0 else rows
    out = pl.pallas_call(
        _sparsemax_rows,
        grid=(rows // r,),
        in_specs=[pl.BlockSpec((r, n), lambda i: (i, 0))],
        out_specs=pl.BlockSpec((r, n), lambda i: (i, 0)),
        out_shape=jax.ShapeDtypeStruct((rows, n), x.dtype),
    )(x)
    return out.reshape(orig_shape)


# speedup vs baseline: 1.4138x; 1.0870x over previous
import jax
import jax.numpy as jnp
from jax.experimental import pallas as pl

_N_ITERS = 6
_BLOCK_ROWS = 256


def _sparsemax_rows(x_ref, o_ref):
    x = x_ref[...]
    m = jnp.max(x, axis=1, keepdims=True)
    s_minus_1 = jnp.sum(x, axis=1, keepdims=True) - 1.0
    lo = m - 1.0
    hi = m
    for _ in range(_N_ITERS):
        mid = 0.5 * (lo + hi)
        smin = jnp.sum(jnp.minimum(x, mid), axis=1, keepdims=True)
        pred = smin <= s_minus_1
        lo = jnp.where(pred, mid, lo)
        hi = jnp.where(pred, hi, mid)
    tau = 0.5 * (lo + hi)
    for _ in range(2):
        mask = x > tau
        k = jnp.sum(mask.astype(jnp.float32), axis=1, keepdims=True)
        s = jnp.sum(jnp.where(mask, x, 0.0), axis=1, keepdims=True)
        tau = (s - 1.0) / k
    o_ref[...] = jnp.maximum(x - tau, 0.0)


def kernel(input):
    orig_shape = input.shape
    n = orig_shape[-1]
    x = input.reshape(-1, n)
    rows = x.shape[0]
    r = _BLOCK_ROWS if rows % _BLOCK_ROWS == 0 else rows
    out = pl.pallas_call(
        _sparsemax_rows,
        grid=(rows // r,),
        in_specs=[pl.BlockSpec((r, n), lambda i: (i, 0))],
        out_specs=pl.BlockSpec((r, n), lambda i: (i, 0)),
        out_shape=jax.ShapeDtypeStruct((rows, n), x.dtype),
    )(x)
    return out.reshape(orig_shape)


# speedup vs baseline: 1.4721x; 1.0412x over previous
import jax
import jax.numpy as jnp
from jax.experimental import pallas as pl

_N_ITERS = 5
_BLOCK_ROWS = 256


def _sparsemax_rows(x_ref, o_ref):
    x = x_ref[...]
    m = jnp.max(x, axis=1, keepdims=True)
    s_minus_1 = jnp.sum(x, axis=1, keepdims=True) - 1.0
    lo = m - 1.0
    hi = m
    for _ in range(_N_ITERS):
        mid = 0.5 * (lo + hi)
        smin = jnp.sum(jnp.minimum(x, mid), axis=1, keepdims=True)
        pred = smin <= s_minus_1
        lo = jnp.where(pred, mid, lo)
        hi = jnp.where(pred, hi, mid)
    tau = 0.5 * (lo + hi)
    for _ in range(2):
        mask = x > tau
        k = jnp.sum(mask.astype(jnp.float32), axis=1, keepdims=True)
        s = jnp.sum(jnp.where(mask, x, 0.0), axis=1, keepdims=True)
        tau = (s - 1.0) / k
    o_ref[...] = jnp.maximum(x - tau, 0.0)


def kernel(input):
    orig_shape = input.shape
    n = orig_shape[-1]
    x = input.reshape(-1, n)
    rows = x.shape[0]
    r = _BLOCK_ROWS if rows % _BLOCK_ROWS == 0 else rows
    out = pl.pallas_call(
        _sparsemax_rows,
        grid=(rows // r,),
        in_specs=[pl.BlockSpec((r, n), lambda i: (i, 0))],
        out_specs=pl.BlockSpec((r, n), lambda i: (i, 0)),
        out_shape=jax.ShapeDtypeStruct((rows, n), x.dtype),
    )(x)
    return out.reshape(orig_shape)
